# Initial kernel scaffold; baseline (speedup 1.0000x reference)
import jax
import jax.numpy as jnp
from jax.experimental import pallas as pl


def kernel(src, tgt, params):
    raise NotImplementedError("write your pallas kernel here")


# speedup vs baseline: 3.9272x; 3.9272x over previous
import jax
import jax.numpy as jnp
from jax.experimental import pallas as pl
from jax.experimental.pallas import tpu as pltpu
from jax.experimental.pallas import tpu_sc as plsc

F32 = jnp.float32
S = 4096
D = 512
H = 8
DH = 64
NB = 64
NH = 4
CH = 64
NC = S // CH
ROWS = NH * H * S


def _vmesh():
    return plsc.VectorSubcoreMesh(core_axis_name="core", subcore_axis_name="subcore")


def _sc_gather(table, idx, width, window):
    n = idx.shape[1]

    @pl.kernel(out_type=jax.ShapeDtypeStruct((n, width), table.dtype),
               mesh=_vmesh())
    def k(x_hbm, i_hbm, o_hbm):
        def body(i_vmem, o_vmem):
            pltpu.sync_copy(x_hbm.at[i_vmem.at[0]], o_vmem)

        pltpu.emit_pipeline(
            body,
            grid=(n // window,),
            in_specs=[pl.BlockSpec((1, window), lambda i: (0, i))],
            out_specs=[pl.BlockSpec((window, width), lambda i: (i, 0))],
            core_axis_name=("core", "subcore"),
            dimension_semantics=(pltpu.PARALLEL,),
        )(i_hbm, o_hbm)

    return k(table, idx)


def _sc_scatter(src, idx, out_rows, width, nreps, window):
    nrows = src.shape[0]
    nwin = nrows // window

    @pl.kernel(out_type=jax.ShapeDtypeStruct((out_rows, width), src.dtype),
               mesh=_vmesh())
    def k(x_hbm, i_hbm, o_hbm):
        def body(x_vmem, i_vmem):
            pltpu.sync_copy(x_vmem, o_hbm.at[i_vmem.at[0]])

        pltpu.emit_pipeline(
            body,
            grid=(nreps, nwin),
            in_specs=[
                pl.BlockSpec((window, width), lambda r, c: (c, 0)),
                pl.BlockSpec((1, window), lambda r, c: (0, r * nwin + c)),
            ],
            out_specs=[],
            core_axis_name=("core", "subcore"),
            dimension_semantics=(pltpu.PARALLEL, pltpu.PARALLEL),
        )(x_hbm, i_hbm)

    return k(src, idx)


def _add(a, b):
    def body(a_ref, b_ref, o_ref):
        o_ref[...] = a_ref[...] + b_ref[...]

    bs = 1024
    return pl.pallas_call(
        body,
        grid=(S // bs,),
        in_specs=[pl.BlockSpec((bs, D), lambda i: (i, 0)),
                  pl.BlockSpec((bs, D), lambda i: (i, 0))],
        out_specs=pl.BlockSpec((bs, D), lambda i: (i, 0)),
        out_shape=jax.ShapeDtypeStruct((S, D), F32),
    )(a, b)


def _layer_norm(x, g, b):
    m = jnp.mean(x, axis=1, keepdims=True)
    xc = x - m
    v = jnp.mean(xc * xc, axis=1, keepdims=True)
    return xc * jax.lax.rsqrt(v + 1e-5) * g + b


def _qkv(x, g, b, wqk, wv):
    bs = 512

    def body(x_ref, g_ref, b_ref, wqk_ref, wv_ref, o_ref):
        nx = _layer_norm(x_ref[...], g_ref[...], b_ref[...])
        qk = jnp.dot(nx, wqk_ref[...], preferred_element_type=F32)
        v = jnp.dot(nx, wv_ref[...], preferred_element_type=F32)
        packed = jnp.concatenate(
            [qk.reshape(bs, H, DH), v.reshape(bs, H, DH)], axis=2)
        o_ref[...] = packed.reshape(bs, H * 2 * DH)

    return pl.pallas_call(
        body,
        grid=(S // bs,),
        in_specs=[pl.BlockSpec((bs, D), lambda i: (i, 0)),
                  pl.BlockSpec((1, D), lambda i: (0, 0)),
                  pl.BlockSpec((1, D), lambda i: (0, 0)),
                  pl.BlockSpec((D, D), lambda i: (0, 0)),
                  pl.BlockSpec((D, D), lambda i: (0, 0))],
        out_specs=pl.BlockSpec((bs, H * 2 * DH), lambda i: (i, 0)),
        out_shape=jax.ShapeDtypeStruct((S, H * 2 * DH), F32),
    )(x, g.reshape(1, D), b.reshape(1, D), wqk, wv)


def _slot_idx(qk2d, rot):
    def body(qk_ref, rot_ref, idx_ref):
        r = pl.program_id(0)
        lane = jax.lax.broadcasted_iota(jnp.int32, (S, NB), 1)
        cols = []
        for h in range(H):
            qh = qk_ref[:, h * 2 * DH:h * 2 * DH + DH]
            pr = jnp.dot(qh, rot_ref[0, h], preferred_element_type=F32)
            pm = jnp.concatenate([pr, -pr], axis=1)
            mx = jnp.max(pm, axis=1, keepdims=True)
            bid = jnp.min(jnp.where(pm >= mx, lane, NB), axis=1,
                          keepdims=True)
            a = (lane == bid).astype(F32)
            g = a
            k = 1
            while k < S:
                g = g + jnp.concatenate(
                    [jnp.zeros((k, NB), F32), g[:S - k, :]], axis=0)
                k *= 2
            tot = g[S - 1:S, :]
            st = tot
            k = 1
            while k < NB:
                st = st + jnp.concatenate(
                    [jnp.zeros((1, k), F32), st[:, :NB - k]], axis=1)
                k *= 2
            starts = st - tot
            slot = jnp.sum((starts + g - 1.0) * a, axis=1, keepdims=True)
            cols.append(slot.astype(jnp.int32) + (r * (H * S) + h * S))
        idx_ref[...] = jnp.concatenate(cols, axis=1)[None]

    return pl.pallas_call(
        body,
        grid=(NH,),
        in_specs=[pl.BlockSpec((S, H * 2 * DH), lambda r: (0, 0)),
                  pl.BlockSpec((1, H, DH, NB // 2), lambda r: (r, 0, 0, 0))],
        out_specs=pl.BlockSpec((1, S, H), lambda r: (r, 0, 0)),
        out_shape=jax.ShapeDtypeStruct((NH, S, H), jnp.int32),
    )(qk2d, rot)


def _attn(sqkv):
    W = 2 * DH

    def body(qv_ref, o_ref):
        zpad = jnp.zeros((CH, DH), F32)

        def chunk(c, carry):
            pc = jax.lax.rem(c + NC - 1, NC)
            cur = qv_ref[0, pl.ds(c * CH, CH), :]
            prev = qv_ref[0, pl.ds(pc * CH, CH), :]
            qc = cur[:, :DH]
            kc = jnp.concatenate([prev[:, :DH], cur[:, :DH]], axis=0)
            vc = jnp.concatenate([prev[:, DH:], cur[:, DH:]], axis=0)
            s = jax.lax.dot_general(qc, kc, (((1,), (1,)), ((), ())),
                                    preferred_element_type=F32) * 0.125
            s = s - jnp.max(s, axis=1, keepdims=True)
            e = jnp.exp(s)
            p = e / jnp.sum(e, axis=1, keepdims=True)
            oc = jnp.dot(p, vc, preferred_element_type=F32)
            o_ref[0, pl.ds(c * CH, CH), :] = jnp.concatenate(
                [oc, zpad], axis=1)
            return carry

        jax.lax.fori_loop(0, NC, chunk, 0)

    n = NH * H
    return pl.pallas_call(
        body,
        grid=(n,),
        in_specs=[pl.BlockSpec((1, S, W), lambda i: (i, 0, 0))],
        out_specs=pl.BlockSpec((1, S, W), lambda i: (i, 0, 0)),
        out_shape=jax.ShapeDtypeStruct((n, S, W), F32),
    )(sqkv)


def _attn_out(o4, x1, wo):
    bs = 1024

    def body(o_ref, x1_ref, wo_ref, y_ref):
        acc = (o_ref[0] + o_ref[1] + o_ref[2] + o_ref[3]) * 0.25
        acc = acc.reshape(bs, H, 2 * DH)[:, :, :DH].reshape(bs, D)
        y_ref[...] = x1_ref[...] + jnp.dot(acc, wo_ref[...],
                                           preferred_element_type=F32)

    return pl.pallas_call(
        body,
        grid=(S // bs,),
        in_specs=[pl.BlockSpec((NH, bs, H * 2 * DH), lambda i: (0, i, 0)),
                  pl.BlockSpec((bs, D), lambda i: (i, 0)),
                  pl.BlockSpec((D, D), lambda i: (0, 0))],
        out_specs=pl.BlockSpec((bs, D), lambda i: (i, 0)),
        out_shape=jax.ShapeDtypeStruct((S, D), F32),
    )(o4, x1, wo)


def _ffn(y1, g, b, w1, b1, w2, b2, x2):
    bs = 512
    dff = w1.shape[1]

    def body(y1_ref, g_ref, b_ref, w1_ref, b1_ref, w2_ref, b2_ref, x2_ref,
             o_ref):
        nx = _layer_norm(y1_ref[...], g_ref[...], b_ref[...])
        hh = jnp.maximum(
            jnp.dot(nx, w1_ref[...], preferred_element_type=F32) + b1_ref[...],
            0.0)
        o_ref[...] = x2_ref[...] + (
            jnp.dot(hh, w2_ref[...], preferred_element_type=F32) + b2_ref[...])

    return pl.pallas_call(
        body,
        grid=(S // bs,),
        in_specs=[pl.BlockSpec((bs, D), lambda i: (i, 0)),
                  pl.BlockSpec((1, D), lambda i: (0, 0)),
                  pl.BlockSpec((1, D), lambda i: (0, 0)),
                  pl.BlockSpec((D, dff), lambda i: (0, 0)),
                  pl.BlockSpec((1, dff), lambda i: (0, 0)),
                  pl.BlockSpec((dff, D), lambda i: (0, 0)),
                  pl.BlockSpec((1, D), lambda i: (0, 0)),
                  pl.BlockSpec((bs, D), lambda i: (i, 0))],
        out_specs=pl.BlockSpec((bs, D), lambda i: (i, 0)),
        out_shape=jax.ShapeDtypeStruct((S, D), F32),
    )(y1, g.reshape(1, D), b.reshape(1, D), w1, b1.reshape(1, dff), w2,
      b2.reshape(1, D), x2)


def _logits(x1, x2, wout, bout):
    bs = 512
    vb = 1024
    vocab = wout.shape[1]

    def body(x1_ref, x2_ref, w_ref, b_ref, o_ref):
        xm = (x1_ref[...] + x2_ref[...]) * 0.5
        o_ref[...] = jnp.dot(xm, w_ref[...],
                             preferred_element_type=F32) + b_ref[...]

    return pl.pallas_call(
        body,
        grid=(S // bs, vocab // vb),
        in_specs=[pl.BlockSpec((bs, D), lambda i, j: (i, 0)),
                  pl.BlockSpec((bs, D), lambda i, j: (i, 0)),
                  pl.BlockSpec((D, vb), lambda i, j: (0, j)),
                  pl.BlockSpec((1, vb), lambda i, j: (0, j))],
        out_specs=pl.BlockSpec((bs, vb), lambda i, j: (i, j)),
        out_shape=jax.ShapeDtypeStruct((S, vocab), F32),
    )(x1, x2, wout, bout.reshape(1, vocab))


def kernel(src, tgt, params):
    p = params
    ids = tgt.reshape(S).astype(jnp.int32)
    ids2 = jnp.stack([ids * 2, ids * 2 + 1], axis=-1).reshape(1, 2 * S)
    half = p['dec_emb'].reshape(-1, D // 2)
    emb_rows = _sc_gather(half, ids2, D // 2, window=128).reshape(S, D)
    x = _add(emb_rows, p['dec_pos'].reshape(-1, D)[:S, :])
    x1, x2 = x, jnp.zeros_like(x)
    for lp in p['dec_layers']:
        qkv = _qkv(x2, lp['ln1_g'], lp['ln1_b'], lp['Wqk'], lp['Wv'])
        idx = _slot_idx(qkv, lp['rot']).reshape(1, ROWS)
        sqkv = _sc_scatter(qkv.reshape(S * H, 2 * DH), idx, ROWS, 2 * DH,
                           nreps=NH, window=128)
        so = _attn(sqkv.reshape(NH * H, S, 2 * DH))
        o = _sc_gather(so.reshape(ROWS, 2 * DH), idx, 2 * DH, window=128)
        y1 = _attn_out(o.reshape(NH, S, H * 2 * DH), x1, lp['Wo'])
        y2 = _ffn(y1, lp['ln2_g'], lp['ln2_b'], lp['W1'], lp['b1'],
                  lp['W2'], lp['b2'], x2)
        x1, x2 = y1, y2
    out = _logits(x1, x2, p['Wout'], p['bout'])
    return out.reshape(1, S, -1)


# speedup vs baseline: 6.9981x; 1.7820x over previous
import jax
import jax.numpy as jnp
from jax.experimental import pallas as pl
from jax.experimental.pallas import tpu as pltpu
from jax.experimental.pallas import tpu_sc as plsc

F32 = jnp.float32
BF16 = jnp.bfloat16


def _bdot(a, b):
    return jnp.dot(a.astype(BF16), b.astype(BF16), preferred_element_type=F32)

S = 4096
D = 512
H = 8
DH = 64
NB = 64
NH = 4
CH = 64
NC = S // CH
ROWS = NH * H * S


def _vmesh():
    return plsc.VectorSubcoreMesh(core_axis_name="core", subcore_axis_name="subcore")


def _sc_gather(table, idx, width, window):
    n = idx.shape[1]

    @pl.kernel(out_type=jax.ShapeDtypeStruct((n, width), table.dtype),
               mesh=_vmesh())
    def k(x_hbm, i_hbm, o_hbm):
        def body(i_vmem, o_vmem):
            pltpu.sync_copy(x_hbm.at[i_vmem.at[0]], o_vmem)

        pltpu.emit_pipeline(
            body,
            grid=(n // window,),
            in_specs=[pl.BlockSpec((1, window), lambda i: (0, i))],
            out_specs=[pl.BlockSpec((window, width), lambda i: (i, 0))],
            core_axis_name=("core", "subcore"),
            dimension_semantics=(pltpu.PARALLEL,),
        )(i_hbm, o_hbm)

    return k(table, idx)


def _sc_scatter(src, idx, out_rows, width, nreps, window):
    nrows = src.shape[0]
    nwin = nrows // window

    @pl.kernel(out_type=jax.ShapeDtypeStruct((out_rows, width), src.dtype),
               mesh=_vmesh())
    def k(x_hbm, i_hbm, o_hbm):
        def body(x_vmem, i_vmem):
            pltpu.sync_copy(x_vmem, o_hbm.at[i_vmem.at[0]])

        pltpu.emit_pipeline(
            body,
            grid=(nreps, nwin),
            in_specs=[
                pl.BlockSpec((window, width), lambda r, c: (c, 0)),
                pl.BlockSpec((1, window), lambda r, c: (0, r * nwin + c)),
            ],
            out_specs=[],
            core_axis_name=("core", "subcore"),
            dimension_semantics=(pltpu.PARALLEL, pltpu.PARALLEL),
        )(x_hbm, i_hbm)

    return k(src, idx)


def _add(a, b):
    def body(a_ref, b_ref, o_ref):
        o_ref[...] = a_ref[...] + b_ref[...]

    bs = 1024
    return pl.pallas_call(
        body,
        grid=(S // bs,),
        in_specs=[pl.BlockSpec((bs, D), lambda i: (i, 0)),
                  pl.BlockSpec((bs, D), lambda i: (i, 0))],
        out_specs=pl.BlockSpec((bs, D), lambda i: (i, 0)),
        out_shape=jax.ShapeDtypeStruct((S, D), F32),
    )(a, b)


def _layer_norm(x, g, b):
    m = jnp.mean(x, axis=1, keepdims=True)
    xc = x - m
    v = jnp.mean(xc * xc, axis=1, keepdims=True)
    return xc * jax.lax.rsqrt(v + 1e-5) * g + b


def _qkv(x, g, b, wqk, wv):
    bs = 512

    def body(x_ref, g_ref, b_ref, wqk_ref, wv_ref, o_ref):
        nx = _layer_norm(x_ref[...], g_ref[...], b_ref[...])
        qk = jnp.dot(nx, wqk_ref[...], preferred_element_type=F32)
        v = jnp.dot(nx, wv_ref[...], preferred_element_type=F32)
        packed = jnp.concatenate(
            [qk.reshape(bs, H, DH), v.reshape(bs, H, DH)], axis=2)
        o_ref[...] = packed.reshape(bs, H * 2 * DH)

    return pl.pallas_call(
        body,
        grid=(S // bs,),
        in_specs=[pl.BlockSpec((bs, D), lambda i: (i, 0)),
                  pl.BlockSpec((1, D), lambda i: (0, 0)),
                  pl.BlockSpec((1, D), lambda i: (0, 0)),
                  pl.BlockSpec((D, D), lambda i: (0, 0)),
                  pl.BlockSpec((D, D), lambda i: (0, 0))],
        out_specs=pl.BlockSpec((bs, H * 2 * DH), lambda i: (i, 0)),
        out_shape=jax.ShapeDtypeStruct((S, H * 2 * DH), F32),
    )(x, g.reshape(1, D), b.reshape(1, D), wqk, wv)


def _slot_idx(qk2d, rot):
    def body(qk_ref, rot_ref, idx_ref):
        r = pl.program_id(0)
        lane = jax.lax.broadcasted_iota(jnp.int32, (S, NB), 1)
        cols = []
        for h in range(H):
            qh = qk_ref[:, h * 2 * DH:h * 2 * DH + DH]
            pr = jnp.dot(qh, rot_ref[0, h], preferred_element_type=F32)
            pm = jnp.concatenate([pr, -pr], axis=1)
            mx = jnp.max(pm, axis=1, keepdims=True)
            bid = jnp.min(jnp.where(pm >= mx, lane, NB), axis=1,
                          keepdims=True)
            a = (lane == bid).astype(F32)
            g = a
            k = 1
            while k < S:
                g = g + jnp.concatenate(
                    [jnp.zeros((k, NB), F32), g[:S - k, :]], axis=0)
                k *= 2
            tot = g[S - 1:S, :]
            st = tot
            k = 1
            while k < NB:
                st = st + jnp.concatenate(
                    [jnp.zeros((1, k), F32), st[:, :NB - k]], axis=1)
                k *= 2
            starts = st - tot
            slot = jnp.sum((starts + g - 1.0) * a, axis=1, keepdims=True)
            cols.append(slot.astype(jnp.int32) + (r * (H * S) + h * S))
        idx_ref[...] = jnp.concatenate(cols, axis=1)[None]

    return pl.pallas_call(
        body,
        grid=(NH,),
        in_specs=[pl.BlockSpec((S, H * 2 * DH), lambda r: (0, 0)),
                  pl.BlockSpec((1, H, DH, NB // 2), lambda r: (r, 0, 0, 0))],
        out_specs=pl.BlockSpec((1, S, H), lambda r: (r, 0, 0)),
        out_shape=jax.ShapeDtypeStruct((NH, S, H), jnp.int32),
    )(qk2d, rot)


def _attn(sqkv):
    W = 2 * DH
    QB = 4 * CH
    KB = QB + CH
    NSTEP = S // QB

    def body(qv_ref, o_ref):
        zpad = jnp.zeros((QB, DH), F32)
        qa = jax.lax.broadcasted_iota(jnp.int32, (QB, KB), 0) // CH
        kb = jax.lax.broadcasted_iota(jnp.int32, (QB, KB), 1) // CH
        band = (kb == qa) | (kb == qa + 1)
        neg = jnp.float32(-1e30)

        def step(c0, carry):
            pstart = jax.lax.rem(c0 * QB + (S - CH), S)
            cur = qv_ref[0, pl.ds(c0 * QB, QB), :]
            prev = qv_ref[0, pl.ds(pstart, CH), :]
            qc = cur[:, :DH].astype(BF16)
            kc = jnp.concatenate([prev[:, :DH], cur[:, :DH]],
                                 axis=0).astype(BF16)
            vc = jnp.concatenate([prev[:, DH:], cur[:, DH:]],
                                 axis=0).astype(BF16)
            s = jax.lax.dot_general(qc, kc, (((1,), (1,)), ((), ())),
                                    preferred_element_type=F32) * 0.125
            s = jnp.where(band, s, neg)
            s = s - jnp.max(s, axis=1, keepdims=True)
            e = jnp.exp(s)
            p = e / jnp.sum(e, axis=1, keepdims=True)
            oc = jnp.dot(p.astype(BF16), vc, preferred_element_type=F32)
            o_ref[0, pl.ds(c0 * QB, QB), :] = jnp.concatenate(
                [oc, zpad], axis=1)
            return carry

        jax.lax.fori_loop(0, NSTEP, step, 0)

    n = NH * H
    return pl.pallas_call(
        body,
        grid=(n,),
        in_specs=[pl.BlockSpec((1, S, W), lambda i: (i, 0, 0))],
        out_specs=pl.BlockSpec((1, S, W), lambda i: (i, 0, 0)),
        out_shape=jax.ShapeDtypeStruct((n, S, W), F32),
    )(sqkv)


def _attn_out(o4, x1, wo):
    bs = 1024

    def body(o_ref, x1_ref, wo_ref, y_ref):
        acc = (o_ref[0] + o_ref[1] + o_ref[2] + o_ref[3]) * 0.25
        acc = acc.reshape(bs, H, 2 * DH)[:, :, :DH].reshape(bs, D)
        y_ref[...] = x1_ref[...] + jnp.dot(acc, wo_ref[...],
                                           preferred_element_type=F32)

    return pl.pallas_call(
        body,
        grid=(S // bs,),
        in_specs=[pl.BlockSpec((NH, bs, H * 2 * DH), lambda i: (0, i, 0)),
                  pl.BlockSpec((bs, D), lambda i: (i, 0)),
                  pl.BlockSpec((D, D), lambda i: (0, 0))],
        out_specs=pl.BlockSpec((bs, D), lambda i: (i, 0)),
        out_shape=jax.ShapeDtypeStruct((S, D), F32),
    )(o4, x1, wo)


def _ffn(y1, g, b, w1, b1, w2, b2, x2):
    bs = 512
    dff = w1.shape[1]

    def body(y1_ref, g_ref, b_ref, w1_ref, b1_ref, w2_ref, b2_ref, x2_ref,
             o_ref):
        nx = _layer_norm(y1_ref[...], g_ref[...], b_ref[...])
        hh = jnp.maximum(
            jnp.dot(nx, w1_ref[...], preferred_element_type=F32) + b1_ref[...],
            0.0)
        o_ref[...] = x2_ref[...] + (
            jnp.dot(hh, w2_ref[...], preferred_element_type=F32) + b2_ref[...])

    return pl.pallas_call(
        body,
        grid=(S // bs,),
        in_specs=[pl.BlockSpec((bs, D), lambda i: (i, 0)),
                  pl.BlockSpec((1, D), lambda i: (0, 0)),
                  pl.BlockSpec((1, D), lambda i: (0, 0)),
                  pl.BlockSpec((D, dff), lambda i: (0, 0)),
                  pl.BlockSpec((1, dff), lambda i: (0, 0)),
                  pl.BlockSpec((dff, D), lambda i: (0, 0)),
                  pl.BlockSpec((1, D), lambda i: (0, 0)),
                  pl.BlockSpec((bs, D), lambda i: (i, 0))],
        out_specs=pl.BlockSpec((bs, D), lambda i: (i, 0)),
        out_shape=jax.ShapeDtypeStruct((S, D), F32),
    )(y1, g.reshape(1, D), b.reshape(1, D), w1, b1.reshape(1, dff), w2,
      b2.reshape(1, D), x2)


def _logits(x1, x2, wout, bout):
    bs = 512
    vb = 1024
    vocab = wout.shape[1]

    def body(x1_ref, x2_ref, w_ref, b_ref, o_ref):
        xm = (x1_ref[...] + x2_ref[...]) * 0.5
        o_ref[...] = _bdot(xm, w_ref[...]) + b_ref[...]

    return pl.pallas_call(
        body,
        grid=(S // bs, vocab // vb),
        in_specs=[pl.BlockSpec((bs, D), lambda i, j: (i, 0)),
                  pl.BlockSpec((bs, D), lambda i, j: (i, 0)),
                  pl.BlockSpec((D, vb), lambda i, j: (0, j)),
                  pl.BlockSpec((1, vb), lambda i, j: (0, j))],
        out_specs=pl.BlockSpec((bs, vb), lambda i, j: (i, j)),
        out_shape=jax.ShapeDtypeStruct((S, vocab), F32),
    )(x1, x2, wout, bout.reshape(1, vocab))


def kernel(src, tgt, params):
    p = params
    ids = tgt.reshape(S).astype(jnp.int32)
    ids2 = jnp.stack([ids * 2, ids * 2 + 1], axis=-1).reshape(1, 2 * S)
    half = p['dec_emb'].reshape(-1, D // 2)
    emb_rows = _sc_gather(half, ids2, D // 2, window=128).reshape(S, D)
    x = _add(emb_rows, p['dec_pos'].reshape(-1, D)[:S, :])
    x1, x2 = x, jnp.zeros_like(x)
    for lp in p['dec_layers']:
        qkv = _qkv(x2, lp['ln1_g'], lp['ln1_b'], lp['Wqk'], lp['Wv'])
        idx = _slot_idx(qkv, lp['rot']).reshape(1, ROWS)
        sqkv = _sc_scatter(qkv.reshape(S * H, 2 * DH), idx, ROWS, 2 * DH,
                           nreps=NH, window=128)
        so = _attn(sqkv.reshape(NH * H, S, 2 * DH))
        o = _sc_gather(so.reshape(ROWS, 2 * DH), idx, 2 * DH, window=128)
        y1 = _attn_out(o.reshape(NH, S, H * 2 * DH), x1, lp['Wo'])
        y2 = _ffn(y1, lp['ln2_g'], lp['ln2_b'], lp['W1'], lp['b1'],
                  lp['W2'], lp['b2'], x2)
        x1, x2 = y1, y2
    out = _logits(x1, x2, p['Wout'], p['bout'])
    return out.reshape(1, S, -1)


# speedup vs baseline: 11.9082x; 1.7016x over previous
import jax
import jax.numpy as jnp
from jax.experimental import pallas as pl
from jax.experimental.pallas import tpu as pltpu
from jax.experimental.pallas import tpu_sc as plsc

F32 = jnp.float32
BF16 = jnp.bfloat16


def _bdot(a, b):
    return jnp.dot(a.astype(BF16), b.astype(BF16), preferred_element_type=F32)

S = 4096
D = 512
H = 8
DH = 64
NB = 64
NH = 4
CH = 64
NC = S // CH
ROWS = NH * H * S


def _vmesh():
    return plsc.VectorSubcoreMesh(core_axis_name="core", subcore_axis_name="subcore")


def _sc_gather(table, idx, width, window):
    n = idx.shape[1]

    @pl.kernel(out_type=jax.ShapeDtypeStruct((n, width), table.dtype),
               mesh=_vmesh())
    def k(x_hbm, i_hbm, o_hbm):
        def body(i_vmem, o_vmem):
            pltpu.sync_copy(x_hbm.at[i_vmem.at[0]], o_vmem)

        pltpu.emit_pipeline(
            body,
            grid=(n // window,),
            in_specs=[pl.BlockSpec((1, window), lambda i: (0, i))],
            out_specs=[pl.BlockSpec((window, width), lambda i: (i, 0))],
            core_axis_name=("core", "subcore"),
            dimension_semantics=(pltpu.PARALLEL,),
        )(i_hbm, o_hbm)

    return k(table, idx)


def _sc_scatter(src, idx, out_rows, width, nreps, window):
    nrows = src.shape[0]
    nwin = nrows // window

    @pl.kernel(out_type=jax.ShapeDtypeStruct((out_rows, width), src.dtype),
               mesh=_vmesh())
    def k(x_hbm, i_hbm, o_hbm):
        def body(x_vmem, i_vmem):
            pltpu.sync_copy(x_vmem, o_hbm.at[i_vmem.at[0]])

        pltpu.emit_pipeline(
            body,
            grid=(nreps, nwin),
            in_specs=[
                pl.BlockSpec((window, width), lambda r, c: (c, 0)),
                pl.BlockSpec((1, window), lambda r, c: (0, r * nwin + c)),
            ],
            out_specs=[],
            core_axis_name=("core", "subcore"),
            dimension_semantics=(pltpu.PARALLEL, pltpu.PARALLEL),
        )(x_hbm, i_hbm)

    return k(src, idx)


def _add(a, b):
    def body(a_ref, b_ref, o_ref):
        o_ref[...] = a_ref[...] + b_ref[...]

    bs = 1024
    return pl.pallas_call(
        body,
        grid=(S // bs,),
        in_specs=[pl.BlockSpec((bs, D), lambda i: (i, 0)),
                  pl.BlockSpec((bs, D), lambda i: (i, 0))],
        out_specs=pl.BlockSpec((bs, D), lambda i: (i, 0)),
        out_shape=jax.ShapeDtypeStruct((S, D), F32),
    )(a, b)


def _layer_norm(x, g, b):
    m = jnp.mean(x, axis=1, keepdims=True)
    xc = x - m
    v = jnp.mean(xc * xc, axis=1, keepdims=True)
    return xc * jax.lax.rsqrt(v + 1e-5) * g + b


def _qkv(x, g, b, wqk, wv):
    bs = 512

    def body(x_ref, g_ref, b_ref, wqk_ref, wv_ref, o_ref):
        nx = _layer_norm(x_ref[...], g_ref[...], b_ref[...])
        qk = jnp.dot(nx, wqk_ref[...], preferred_element_type=F32)
        v = jnp.dot(nx, wv_ref[...], preferred_element_type=F32)
        packed = jnp.concatenate(
            [qk.reshape(bs, H, DH), v.reshape(bs, H, DH)], axis=2)
        o_ref[...] = packed.reshape(bs, H * 2 * DH)

    return pl.pallas_call(
        body,
        grid=(S // bs,),
        in_specs=[pl.BlockSpec((bs, D), lambda i: (i, 0)),
                  pl.BlockSpec((1, D), lambda i: (0, 0)),
                  pl.BlockSpec((1, D), lambda i: (0, 0)),
                  pl.BlockSpec((D, D), lambda i: (0, 0)),
                  pl.BlockSpec((D, D), lambda i: (0, 0))],
        out_specs=pl.BlockSpec((bs, H * 2 * DH), lambda i: (i, 0)),
        out_shape=jax.ShapeDtypeStruct((S, H * 2 * DH), F32),
    )(x, g.reshape(1, D), b.reshape(1, D), wqk, wv)


def _slot_idx(qk2d, rot):
    def body(qk_ref, rot_ref, idx_ref):
        r = pl.program_id(0)
        lane = jax.lax.broadcasted_iota(jnp.int32, (S, NB), 1)
        cols = []
        for h in range(H):
            qh = qk_ref[:, h * 2 * DH:h * 2 * DH + DH]
            pr = jnp.dot(qh, rot_ref[0, h], preferred_element_type=F32)
            pm = jnp.concatenate([pr, -pr], axis=1)
            mx = jnp.max(pm, axis=1, keepdims=True)
            bid = jnp.min(jnp.where(pm >= mx, lane, NB), axis=1,
                          keepdims=True)
            a = (lane == bid).astype(F32)
            g = a
            k = 1
            while k < S:
                g = g + jnp.concatenate(
                    [jnp.zeros((k, NB), F32), g[:S - k, :]], axis=0)
                k *= 2
            tot = g[S - 1:S, :]
            st = tot
            k = 1
            while k < NB:
                st = st + jnp.concatenate(
                    [jnp.zeros((1, k), F32), st[:, :NB - k]], axis=1)
                k *= 2
            starts = st - tot
            slot = jnp.sum((starts + g - 1.0) * a, axis=1, keepdims=True)
            cols.append(slot.astype(jnp.int32) + (r * (H * S) + h * S))
        idx_ref[...] = jnp.concatenate(cols, axis=1)[None]

    return pl.pallas_call(
        body,
        grid=(NH,),
        in_specs=[pl.BlockSpec((S, H * 2 * DH), lambda r: (0, 0)),
                  pl.BlockSpec((1, H, DH, NB // 2), lambda r: (r, 0, 0, 0))],
        out_specs=pl.BlockSpec((1, S, H), lambda r: (r, 0, 0)),
        out_shape=jax.ShapeDtypeStruct((NH, S, H), jnp.int32),
    )(qk2d, rot)


def _attn(sqkv):
    W = 2 * DH
    QB = 4 * CH
    KB = QB + CH
    NSTEP = S // QB

    def body(qv_ref, o_ref):
        zpad = jnp.zeros((QB, DH), F32)
        qa = jax.lax.broadcasted_iota(jnp.int32, (QB, KB), 0) // CH
        kb = jax.lax.broadcasted_iota(jnp.int32, (QB, KB), 1) // CH
        band = (kb == qa) | (kb == qa + 1)
        neg = jnp.float32(-1e30)

        def step(c0, carry):
            pstart = jax.lax.rem(c0 * QB + (S - CH), S)
            cur = qv_ref[0, pl.ds(c0 * QB, QB), :]
            prev = qv_ref[0, pl.ds(pstart, CH), :]
            qc = cur[:, :DH].astype(BF16)
            kc = jnp.concatenate([prev[:, :DH], cur[:, :DH]],
                                 axis=0).astype(BF16)
            vc = jnp.concatenate([prev[:, DH:], cur[:, DH:]],
                                 axis=0).astype(BF16)
            s = jax.lax.dot_general(qc, kc, (((1,), (1,)), ((), ())),
                                    preferred_element_type=F32) * 0.125
            s = jnp.where(band, s, neg)
            s = s - jnp.max(s, axis=1, keepdims=True)
            e = jnp.exp(s)
            p = e / jnp.sum(e, axis=1, keepdims=True)
            oc = jnp.dot(p.astype(BF16), vc, preferred_element_type=F32)
            o_ref[0, pl.ds(c0 * QB, QB), :] = jnp.concatenate(
                [oc, zpad], axis=1)
            return carry

        jax.lax.fori_loop(0, NSTEP, step, 0)

    n = NH * H
    return pl.pallas_call(
        body,
        grid=(n,),
        in_specs=[pl.BlockSpec((1, S, W), lambda i: (i, 0, 0))],
        out_specs=pl.BlockSpec((1, S, W), lambda i: (i, 0, 0)),
        out_shape=jax.ShapeDtypeStruct((n, S, W), F32),
    )(sqkv)


def _attn_zero_x2(x1, b, wv, wo):
    bs = 1024

    def body(x1_ref, b_ref, wv_ref, wo_ref, y_ref):
        vrow = jnp.dot(b_ref[...], wv_ref[...], preferred_element_type=F32)
        row = jnp.dot(vrow, wo_ref[...], preferred_element_type=F32)
        y_ref[...] = x1_ref[...] + row

    return pl.pallas_call(
        body,
        grid=(S // bs,),
        in_specs=[pl.BlockSpec((bs, D), lambda i: (i, 0)),
                  pl.BlockSpec((1, D), lambda i: (0, 0)),
                  pl.BlockSpec((D, D), lambda i: (0, 0)),
                  pl.BlockSpec((D, D), lambda i: (0, 0))],
        out_specs=pl.BlockSpec((bs, D), lambda i: (i, 0)),
        out_shape=jax.ShapeDtypeStruct((S, D), F32),
    )(x1, b.reshape(1, D), wv, wo)


def _attn_out(o4, x1, wo):
    bs = 1024

    def body(o_ref, x1_ref, wo_ref, y_ref):
        acc = (o_ref[0] + o_ref[1] + o_ref[2] + o_ref[3]) * 0.25
        acc = acc.reshape(bs, H, 2 * DH)[:, :, :DH].reshape(bs, D)
        y_ref[...] = x1_ref[...] + jnp.dot(acc, wo_ref[...],
                                           preferred_element_type=F32)

    return pl.pallas_call(
        body,
        grid=(S // bs,),
        in_specs=[pl.BlockSpec((NH, bs, H * 2 * DH), lambda i: (0, i, 0)),
                  pl.BlockSpec((bs, D), lambda i: (i, 0)),
                  pl.BlockSpec((D, D), lambda i: (0, 0))],
        out_specs=pl.BlockSpec((bs, D), lambda i: (i, 0)),
        out_shape=jax.ShapeDtypeStruct((S, D), F32),
    )(o4, x1, wo)


def _ffn(y1, g, b, w1, b1, w2, b2, x2):
    bs = 512
    dff = w1.shape[1]

    def body(y1_ref, g_ref, b_ref, w1_ref, b1_ref, w2_ref, b2_ref, x2_ref,
             o_ref):
        nx = _layer_norm(y1_ref[...], g_ref[...], b_ref[...])
        hh = jnp.maximum(
            jnp.dot(nx, w1_ref[...], preferred_element_type=F32) + b1_ref[...],
            0.0)
        o_ref[...] = x2_ref[...] + (
            jnp.dot(hh, w2_ref[...], preferred_element_type=F32) + b2_ref[...])

    return pl.pallas_call(
        body,
        grid=(S // bs,),
        in_specs=[pl.BlockSpec((bs, D), lambda i: (i, 0)),
                  pl.BlockSpec((1, D), lambda i: (0, 0)),
                  pl.BlockSpec((1, D), lambda i: (0, 0)),
                  pl.BlockSpec((D, dff), lambda i: (0, 0)),
                  pl.BlockSpec((1, dff), lambda i: (0, 0)),
                  pl.BlockSpec((dff, D), lambda i: (0, 0)),
                  pl.BlockSpec((1, D), lambda i: (0, 0)),
                  pl.BlockSpec((bs, D), lambda i: (i, 0))],
        out_specs=pl.BlockSpec((bs, D), lambda i: (i, 0)),
        out_shape=jax.ShapeDtypeStruct((S, D), F32),
    )(y1, g.reshape(1, D), b.reshape(1, D), w1, b1.reshape(1, dff), w2,
      b2.reshape(1, D), x2)


def _logits(x1, x2, wout, bout):
    bs = 512
    vb = 1024
    vocab = wout.shape[1]

    def body(x1_ref, x2_ref, w_ref, b_ref, o_ref):
        xm = (x1_ref[...] + x2_ref[...]) * 0.5
        o_ref[...] = _bdot(xm, w_ref[...]) + b_ref[...]

    return pl.pallas_call(
        body,
        grid=(S // bs, vocab // vb),
        in_specs=[pl.BlockSpec((bs, D), lambda i, j: (i, 0)),
                  pl.BlockSpec((bs, D), lambda i, j: (i, 0)),
                  pl.BlockSpec((D, vb), lambda i, j: (0, j)),
                  pl.BlockSpec((1, vb), lambda i, j: (0, j))],
        out_specs=pl.BlockSpec((bs, vb), lambda i, j: (i, j)),
        out_shape=jax.ShapeDtypeStruct((S, vocab), F32),
    )(x1, x2, wout, bout.reshape(1, vocab))


def kernel(src, tgt, params):
    p = params
    ids = tgt.reshape(S).astype(jnp.int32)
    ids2 = jnp.stack([ids * 2, ids * 2 + 1], axis=-1).reshape(1, 2 * S)
    half = p['dec_emb'].reshape(-1, D // 2)
    emb_rows = _sc_gather(half, ids2, D // 2, window=128).reshape(S, D)
    x = _add(emb_rows, p['dec_pos'].reshape(-1, D)[:S, :])
    x1, x2 = x, jnp.zeros_like(x)
    lp0 = p['dec_layers'][0]
    y1 = _attn_zero_x2(x1, lp0['ln1_b'], lp0['Wv'], lp0['Wo'])
    y2 = _ffn(y1, lp0['ln2_g'], lp0['ln2_b'], lp0['W1'], lp0['b1'],
              lp0['W2'], lp0['b2'], x2)
    x1, x2 = y1, y2
    for lp in p['dec_layers'][1:]:
        qkv = _qkv(x2, lp['ln1_g'], lp['ln1_b'], lp['Wqk'], lp['Wv'])
        idx = _slot_idx(qkv, lp['rot']).reshape(1, ROWS)
        sqkv = _sc_scatter(qkv.reshape(S * H, 2 * DH), idx, ROWS, 2 * DH,
                           nreps=NH, window=128)
        so = _attn(sqkv.reshape(NH * H, S, 2 * DH))
        o = _sc_gather(so.reshape(ROWS, 2 * DH), idx, 2 * DH, window=128)
        y1 = _attn_out(o.reshape(NH, S, H * 2 * DH), x1, lp['Wo'])
        y2 = _ffn(y1, lp['ln2_g'], lp['ln2_b'], lp['W1'], lp['b1'],
                  lp['W2'], lp['b2'], x2)
        x1, x2 = y1, y2
    out = _logits(x1, x2, p['Wout'], p['bout'])
    return out.reshape(1, S, -1)
